# baseline (speedup 1.0000x reference)
import functools

import jax
import jax.numpy as jnp
from jax import lax
from jax.experimental import pallas as pl
from jax.experimental.pallas import tpu as pltpu
from jax.experimental.pallas import tpu_sc as plsc

N = 10000
F = 256
S1 = 16
S2 = 16
B = 128
D1 = 128
D2 = 128
IN_DIM = 1 + F
G = 2 + 2 * S2
M = 2 * S1 * G
BM = B * M
NW = 32
NODES_PER_W = B // NW
SLOTS_PER_W = NODES_PER_W * 2 * S1
EPW = SLOTS_PER_W * G
HCHUNK = 128
NCHUNK = EPW // HCHUNK


_GDN = lax.GatherDimensionNumbers(offset_dims=(), collapsed_slice_dims=(0,),
                                  start_index_map=(0,))


def _vgat(vec, idx):
    return lax.gather(vec, idx[:, None], _GDN, (1,),
                      mode=lax.GatherScatterMode.PROMISE_IN_BOUNDS)


def _sc_body(nid_hbm, tab_hbm, p_hbm,
             h_hbm, wf_hbm, wc_hbm,
             nid_v, hub1_v, p_small, wp_small, p_v,
             hub2_v, n_v, wfv, wcv, rows_v, rows2_v, sem):
    wid = lax.axis_index("s") * 2 + lax.axis_index("c")
    iota = lax.iota(jnp.int32, 16)

    pltpu.sync_copy(nid_hbm, nid_v)
    pltpu.async_copy(tab_hbm.at[nid_v], hub1_v, sem).wait()

    for j in range(NODES_PER_W):
        b = wid * NODES_PER_W + j
        p_small[2 * j] = hub1_v[b, pl.ds(0, 16)]
        p_small[2 * j + 1] = hub1_v[b, pl.ds(16, 16)]
        wp_small[2 * j] = lax.bitcast_convert_type(hub1_v[b, pl.ds(32, 16)], jnp.float32)
        wp_small[2 * j + 1] = lax.bitcast_convert_type(hub1_v[b, pl.ds(48, 16)],
                                                 jnp.float32)
    for r in range(2 * NODES_PER_W):
        p_v[pl.ds(r * 16, 16)] = p_small[r]

    pltpu.async_copy(tab_hbm.at[p_v], hub2_v, sem).wait()

    sh1 = jnp.clip(iota - 1, 0, 15)
    sh2 = jnp.clip(iota - 2, 0, 15)
    lo15 = jnp.clip(iota + 15, 0, 15)
    hi14 = jnp.clip(iota + 14, 0, 15)

    def slot_body(k, carry):
        r = k // 16
        lane = k - r * 16
        blane = iota * 0 + lane
        pb = _vgat(p_small[r], blane)
        wpb = _vgat(wp_small[r], blane)
        ri_s = hub2_v[k, pl.ds(0, 16)]
        ro_s = hub2_v[k, pl.ds(16, 16)]
        ri_a = lax.bitcast_convert_type(hub2_v[k, pl.ds(32, 16)], jnp.float32)
        ro_a = lax.bitcast_convert_type(hub2_v[k, pl.ds(48, 16)], jnp.float32)
        is0 = iota == 0
        is1 = iota == 1
        nA = jnp.where(is0, pb, _vgat(ri_s, sh1))
        nB = jnp.where(is0, _vgat(ri_s, lo15),
                       jnp.where(is1, pb, _vgat(ro_s, sh2)))
        nC = _vgat(ro_s, hi14)
        fA = jnp.where(is0, wpb, _vgat(ri_a, sh1))
        fB = jnp.where(is0, _vgat(ri_a, lo15),
                       jnp.where(is1, wpb, _vgat(ro_a, sh2)))
        fC = _vgat(ro_a, hi14)
        base = k * G
        n_v[pl.ds(base, 16)] = nA
        n_v[pl.ds(base + 16, 16)] = nB
        n_v[pl.ds(base + 32, 16)] = nC
        wfv[pl.ds(base, 16)] = fA
        wfv[pl.ds(base + 16, 16)] = fB
        wfv[pl.ds(base + 32, 16)] = fC
        wcv[pl.ds(base, 16)] = fA * wpb
        wcv[pl.ds(base + 16, 16)] = fB * wpb
        wcv[pl.ds(base + 32, 16)] = fC * wpb
        return carry

    lax.fori_loop(0, SLOTS_PER_W, slot_body, 0)

    base_e = wid * EPW
    pltpu.sync_copy(wfv.at[pl.ds(0, EPW)], wf_hbm.at[pl.ds(base_e, EPW)])
    pltpu.sync_copy(wcv.at[pl.ds(0, EPW)], wc_hbm.at[pl.ds(base_e, EPW)])

    bufs = (rows_v, rows2_v)

    def _gather_start(cix, buf):
        idxs = n_v.at[pl.ds(cix * HCHUNK, HCHUNK)]
        return pltpu.async_copy(p_hbm.at[idxs], buf, sem)

    cp = _gather_start(0, bufs[0])
    for cix in range(NCHUNK):
        cp.wait()
        if cix + 1 < NCHUNK:
            cp = _gather_start(cix + 1, bufs[(cix + 1) % 2])
        pltpu.sync_copy(bufs[cix % 2],
                        h_hbm.at[pl.ds(base_e + cix * HCHUNK, HCHUNK)])


def _sc_gather(node_ids, tab, p_tab):
    mesh = plsc.VectorSubcoreMesh(core_axis_name="c", subcore_axis_name="s",
                                  num_cores=2, num_subcores=16)
    f = pl.kernel(
        _sc_body,
        compiler_params=pltpu.CompilerParams(skip_device_barrier=True),
        out_type=[
            jax.ShapeDtypeStruct((BM, D1), jnp.float32),
            jax.ShapeDtypeStruct((BM,), jnp.float32),
            jax.ShapeDtypeStruct((BM,), jnp.float32),
        ],
        mesh=mesh,
        scratch_types=[
            pltpu.VMEM((B,), jnp.int32),
            pltpu.VMEM((B, 128), jnp.int32),
            pltpu.VMEM((2 * NODES_PER_W, S1), jnp.int32),
            pltpu.VMEM((2 * NODES_PER_W, S1), jnp.float32),
            pltpu.VMEM((SLOTS_PER_W,), jnp.int32),
            pltpu.VMEM((SLOTS_PER_W, 128), jnp.int32),
            pltpu.VMEM((EPW + 16,), jnp.int32),
            pltpu.VMEM((EPW + 16,), jnp.float32),
            pltpu.VMEM((EPW + 16,), jnp.float32),
            pltpu.VMEM((HCHUNK, D1), jnp.float32),
            pltpu.VMEM((HCHUNK, D1), jnp.float32),
            pltpu.SemaphoreType.DMA,
        ],
    )
    return f(node_ids, tab, p_tab)


def _p_body(feat_ref, w_ref, b_ref, out_ref):
    out_ref[...] = (
        jnp.dot(feat_ref[...], w_ref[...], preferred_element_type=jnp.float32)
        + b_ref[...])


def _precompute_p(features, w1f, b1):
    blk = 1000
    return pl.pallas_call(
        _p_body,
        grid=(N // blk,),
        in_specs=[
            pl.BlockSpec((blk, F), lambda i: (i, 0)),
            pl.BlockSpec((F, D1), lambda i: (0, 0)),
            pl.BlockSpec((1, D1), lambda i: (0, 0)),
        ],
        out_specs=pl.BlockSpec((blk, D1), lambda i: (i, 0)),
        out_shape=jax.ShapeDtypeStruct((N, D1), jnp.float32),
    )(features, w1f, b1.reshape(1, D1))


def _dec_body(wd2_ref, wd1_ref, bd2_ref, bd1t_ref, w2_ref, r0_ref,
              wcatt_ref, ccatt_ref, u_ref):
    wd1 = wd1_ref[...]
    bd1t = bd1t_ref[...]
    for g in range(G):
        wcatt_ref[g * IN_DIM:(g + 1) * IN_DIM, :] = lax.dot_general(
            wd1, wd2_ref[:, g * D1:(g + 1) * D1],
            (((0,), (1,)), ((), ())),
            preferred_element_type=jnp.float32)
        cct = lax.dot_general(
            wd1, bd2_ref[:, g * D1:(g + 1) * D1],
            (((0,), (1,)), ((), ())),
            preferred_element_type=jnp.float32) + bd1t
        ccatt_ref[g * IN_DIM:(g + 1) * IN_DIM, :] = jnp.broadcast_to(
            cct, (IN_DIM, D2))
        u_ref[:, g * D1:(g + 1) * D1] = jnp.dot(
            r0_ref[...], w2_ref[g * D1:(g + 1) * D1, :],
            preferred_element_type=jnp.float32)


def _precompute_dec(w_dec2, w_dec1, b_dec2, b_dec1, w_enc2, r0):
    return pl.pallas_call(
        _dec_body,
        in_specs=[
            pl.BlockSpec((D2, G * D1), lambda: (0, 0)),
            pl.BlockSpec((D1, IN_DIM), lambda: (0, 0)),
            pl.BlockSpec((1, G * D1), lambda: (0, 0)),
            pl.BlockSpec((IN_DIM, 1), lambda: (0, 0)),
            pl.BlockSpec((G * D1, D2), lambda: (0, 0)),
            pl.BlockSpec((1, D1), lambda: (0, 0)),
        ],
        out_specs=[
            pl.BlockSpec((G * IN_DIM, D2), lambda: (0, 0)),
            pl.BlockSpec((G * IN_DIM, D2), lambda: (0, 0)),
            pl.BlockSpec((1, G * D1), lambda: (0, 0)),
        ],
        out_shape=[
            jax.ShapeDtypeStruct((G * IN_DIM, D2), jnp.float32),
            jax.ShapeDtypeStruct((G * IN_DIM, D2), jnp.float32),
            jax.ShapeDtypeStruct((1, G * D1), jnp.float32),
        ],
    )(w_dec2, w_dec1, b_dec2.reshape(1, G * D1), b_dec1.reshape(IN_DIM, 1),
      w_enc2, r0.reshape(1, D1))


def _main_body(h_ref, wf_ref, w2_ref, u_ref, b2_ref, h2_ref):
    h2_ref[...] = (
        jnp.dot(h_ref[...].astype(jnp.bfloat16),
                w2_ref[...].astype(jnp.bfloat16),
                preferred_element_type=jnp.float32)
        + jnp.dot(wf_ref[...], u_ref[...], preferred_element_type=jnp.float32)
        + b2_ref[...])


def _main(hr, wfr, w_enc2, u, b_enc2):
    nblk = 16
    rows = B * 2 * S1 // nblk
    return pl.pallas_call(
        _main_body,
        grid=(nblk,),
        in_specs=[
            pl.BlockSpec((rows, G * D1), lambda i: (i, 0)),
            pl.BlockSpec((rows, G), lambda i: (i, 0)),
            pl.BlockSpec((G * D1, D2), lambda i: (0, 0)),
            pl.BlockSpec((G, D2), lambda i: (0, 0)),
            pl.BlockSpec((1, D2), lambda i: (0, 0)),
        ],
        out_specs=pl.BlockSpec((rows, D2), lambda i: (i, 0)),
        out_shape=jax.ShapeDtypeStruct((B * 2 * S1, D2), jnp.float32),
    )(hr, wfr, w_enc2, u, b_enc2.reshape(1, D2))


_SPB = 4
_RBLK = _SPB * G


def _recon_body(h2_ref, wcatt_ref, ccatt_ref, rec_ref):
    i = pl.program_id(0)
    for sl in range(_SPB):
        h2s = h2_ref[:, i * _SPB + sl, :].astype(jnp.bfloat16)
        for g in range(G):
            rec_ref[:, sl * G + g, :] = (
                lax.dot_general(
                    wcatt_ref[pl.ds(g * IN_DIM, IN_DIM), :].astype(
                        jnp.bfloat16),
                    h2s, (((1,), (1,)), ((), ())),
                    preferred_element_type=jnp.float32)
                + ccatt_ref[pl.ds(g * IN_DIM, IN_DIM), :])


def _recon(h2f, wcatt, ccatt):
    nblk = (2 * S1) // _SPB
    return pl.pallas_call(
        _recon_body,
        grid=(nblk,),
        compiler_params=pltpu.CompilerParams(
            vmem_limit_bytes=100 * 1024 * 1024),
        in_specs=[
            pl.BlockSpec((B, 2 * S1, D2), lambda i: (0, 0, 0)),
            pl.BlockSpec((G * IN_DIM, D2), lambda i: (0, 0)),
            pl.BlockSpec((G * IN_DIM, D2), lambda i: (0, 0)),
        ],
        out_specs=pl.BlockSpec((IN_DIM, _RBLK, B), lambda i: (0, i, 0)),
        out_shape=jax.ShapeDtypeStruct((IN_DIM, M, B), jnp.float32),
    )(h2f.reshape(B, 2 * S1, D2), wcatt, ccatt)


def kernel(node_ids, features, in_sample, out_sample, in_sample_amnt,
           out_sample_amnt, W_enc1, b_enc1, W_enc2, b_enc2, W_dec2, b_dec2,
           W_dec1, b_dec1):
    ina = in_sample_amnt[..., 0]
    outa = out_sample_amnt[..., 0]
    r0 = W_enc1[0]

    tab = jnp.concatenate([
        in_sample.astype(jnp.int32),
        out_sample.astype(jnp.int32),
        lax.bitcast_convert_type(ina, jnp.int32),
        lax.bitcast_convert_type(outa, jnp.int32),
        jnp.zeros((N, 64), jnp.int32),
    ], axis=1)

    p_tab = _precompute_p(features, W_enc1[1:], b_enc1)
    wcatt, ccatt, u2 = _precompute_dec(W_dec2, W_dec1, b_dec2, b_dec1,
                                      W_enc2, r0)
    h, wf, wc = _sc_gather(node_ids.astype(jnp.int32), tab, p_tab)

    hr = h.reshape(B * 2 * S1, G * D1)
    wfr = wf.reshape(B * 2 * S1, G)
    h2f = _main(hr, wfr, W_enc2, u2.reshape(G, D1), b_enc2)
    rect = _recon(h2f, wcatt, ccatt)

    return (h2f.reshape(B, 2 * S1, D2),
            jnp.transpose(rect, (2, 1, 0)),
            wc.reshape(B, M))


# speedup vs baseline: 1.3395x; 1.3395x over previous
import functools

import jax
import jax.numpy as jnp
from jax import lax
from jax.experimental import pallas as pl
from jax.experimental.pallas import tpu as pltpu
from jax.experimental.pallas import tpu_sc as plsc

N = 10000
F = 256
S1 = 16
S2 = 16
B = 128
D1 = 128
D2 = 128
IN_DIM = 1 + F
G = 2 + 2 * S2
M = 2 * S1 * G
BM = B * M
NW = 32
NODES_PER_W = B // NW
SLOTS_PER_W = NODES_PER_W * 2 * S1
EPW = SLOTS_PER_W * G
HCHUNK = 128
NCHUNK = EPW // HCHUNK


_GDN = lax.GatherDimensionNumbers(offset_dims=(), collapsed_slice_dims=(0,),
                                  start_index_map=(0,))


def _vgat(vec, idx):
    return lax.gather(vec, idx[:, None], _GDN, (1,),
                      mode=lax.GatherScatterMode.PROMISE_IN_BOUNDS)


def _sc_body(nid_hbm, tab_hbm, p_hbm,
             h_hbm, wf_hbm, wc_hbm,
             nid_v, hub1_v, p_small, wp_small, p_v,
             hub2_v, n_v, wfv, wcv, vidx, vidx2, rows_v, rows2_v, sem, sem2):
    wid = lax.axis_index("s") * 2 + lax.axis_index("c")
    iota = lax.iota(jnp.int32, 16)

    pltpu.sync_copy(nid_hbm, nid_v)
    pltpu.async_copy(tab_hbm.at[nid_v], hub1_v, sem).wait()

    for j in range(NODES_PER_W):
        b = wid * NODES_PER_W + j
        p_small[2 * j] = hub1_v[b, pl.ds(0, 16)]
        p_small[2 * j + 1] = hub1_v[b, pl.ds(32, 16)]
        wp_small[2 * j] = lax.bitcast_convert_type(hub1_v[b, pl.ds(64, 16)],
                                                   jnp.float32)
        wp_small[2 * j + 1] = lax.bitcast_convert_type(
            hub1_v[b, pl.ds(96, 16)], jnp.float32)
    for r in range(2 * NODES_PER_W):
        p_v[pl.ds(r * 16, 16)] = p_small[r]

    pltpu.async_copy(tab_hbm.at[p_v], hub2_v, sem).wait()

    sh1 = jnp.clip(iota - 1, 0, 15)
    sh2 = jnp.clip(iota - 2, 0, 15)
    lo15 = jnp.clip(iota + 15, 0, 15)
    hi14 = jnp.clip(iota + 14, 0, 15)

    def slot_body(k, carry):
        r = k // 16
        lane = k - r * 16
        blane = iota * 0 + lane
        pb = _vgat(p_small[r], blane)
        wpb = _vgat(wp_small[r], blane)
        ri_s = hub2_v[k, pl.ds(0, 16)]
        ro_s = hub2_v[k, pl.ds(32, 16)]
        ri_a = lax.bitcast_convert_type(hub2_v[k, pl.ds(64, 16)], jnp.float32)
        ro_a = lax.bitcast_convert_type(hub2_v[k, pl.ds(96, 16)], jnp.float32)
        is0 = iota == 0
        is1 = iota == 1
        nA = jnp.where(is0, pb, _vgat(ri_s, sh1))
        nB = jnp.where(is0, _vgat(ri_s, lo15),
                       jnp.where(is1, pb, _vgat(ro_s, sh2)))
        nC = _vgat(ro_s, hi14)
        fA = jnp.where(is0, wpb, _vgat(ri_a, sh1))
        fB = jnp.where(is0, _vgat(ri_a, lo15),
                       jnp.where(is1, wpb, _vgat(ro_a, sh2)))
        fC = _vgat(ro_a, hi14)
        base = k * G
        n_v[pl.ds(base, 16)] = nA
        n_v[pl.ds(base + 16, 16)] = nB
        n_v[pl.ds(base + 32, 16)] = nC
        wfv[pl.ds(base, 16)] = fA
        wfv[pl.ds(base + 16, 16)] = fB
        wfv[pl.ds(base + 32, 16)] = fC
        wcv[pl.ds(base, 16)] = fA * wpb
        wcv[pl.ds(base + 16, 16)] = fB * wpb
        wcv[pl.ds(base + 32, 16)] = fC * wpb
        rg = wid * SLOTS_PER_W + k
        vbase = (rg // 8) * (8 * G) + (rg - (rg // 8) * 8)
        vidx[pl.ds(base, 16)] = vbase + 8 * iota
        vidx[pl.ds(base + 16, 16)] = vbase + 8 * (16 + iota)
        vidx[pl.ds(base + 32, 16)] = vbase + 8 * (32 + iota)
        return carry

    lax.fori_loop(0, SLOTS_PER_W, slot_body, 0)

    def vstage(c, carry):
        for j in range(HCHUNK // 16):
            vidx2[c, pl.ds(j * 16, 16)] = vidx[pl.ds(c * HCHUNK + j * 16, 16)]
        return carry

    lax.fori_loop(0, NCHUNK, vstage, 0)

    base_e = wid * EPW
    pltpu.sync_copy(wfv.at[pl.ds(0, EPW)], wf_hbm.at[pl.ds(base_e, EPW)])
    pltpu.sync_copy(wcv.at[pl.ds(0, EPW)], wc_hbm.at[pl.ds(base_e, EPW)])

    bufs = (rows_v, rows2_v)

    def _gather_start(cix, buf):
        idxs = n_v.at[pl.ds(cix * HCHUNK, HCHUNK)]
        return pltpu.async_copy(p_hbm.at[idxs], buf, sem)

    cp = _gather_start(0, bufs[0])
    for cix in range(NCHUNK):
        cp.wait()
        if cix + 1 < NCHUNK:
            cp = _gather_start(cix + 1, bufs[(cix + 1) % 2])
        pltpu.async_copy(bufs[cix % 2], h_hbm.at[vidx2.at[cix]],
                         sem2).wait()


def _sc_gather(node_ids, tab, p_tab):
    mesh = plsc.VectorSubcoreMesh(core_axis_name="c", subcore_axis_name="s",
                                  num_cores=2, num_subcores=16)
    f = pl.kernel(
        _sc_body,
        compiler_params=pltpu.CompilerParams(skip_device_barrier=True),
        out_type=[
            jax.ShapeDtypeStruct((BM, D1), jnp.float32),
            jax.ShapeDtypeStruct((BM,), jnp.float32),
            jax.ShapeDtypeStruct((BM,), jnp.float32),
        ],
        mesh=mesh,
        scratch_types=[
            pltpu.VMEM((B,), jnp.int32),
            pltpu.VMEM((B, 128), jnp.int32),
            pltpu.VMEM((2 * NODES_PER_W, S1), jnp.int32),
            pltpu.VMEM((2 * NODES_PER_W, S1), jnp.float32),
            pltpu.VMEM((SLOTS_PER_W,), jnp.int32),
            pltpu.VMEM((SLOTS_PER_W, 128), jnp.int32),
            pltpu.VMEM((EPW + 16,), jnp.int32),
            pltpu.VMEM((EPW + 16,), jnp.float32),
            pltpu.VMEM((EPW + 16,), jnp.float32),
            pltpu.VMEM((EPW + 16,), jnp.int32),
            pltpu.VMEM((NCHUNK, HCHUNK), jnp.int32),
            pltpu.VMEM((HCHUNK, D1), jnp.float32),
            pltpu.VMEM((HCHUNK, D1), jnp.float32),
            pltpu.SemaphoreType.DMA,
            pltpu.SemaphoreType.DMA,
        ],
    )
    return f(node_ids, tab, p_tab)


_PBLK = 1000


def _p_body(feat_ref, w_ref, b_ref, ins_ref, outs_ref, ina_ref, outa_ref,
            out_ref, tab_ref):
    out_ref[...] = (
        jnp.dot(feat_ref[...], w_ref[...], preferred_element_type=jnp.float32)
        + b_ref[...])
    z = jnp.zeros((_PBLK, S1), jnp.int32)
    tab_ref[...] = jnp.concatenate([
        ins_ref[...], z, outs_ref[...], z,
        lax.bitcast_convert_type(ina_ref[...], jnp.int32), z,
        lax.bitcast_convert_type(outa_ref[...], jnp.int32), z,
    ], axis=1)


def _precompute_p(features, w1f, b1, in_sample, out_sample, ina, outa):
    return pl.pallas_call(
        _p_body,
        grid=(N // _PBLK,),
        in_specs=[
            pl.BlockSpec((_PBLK, F), lambda i: (i, 0)),
            pl.BlockSpec((F, D1), lambda i: (0, 0)),
            pl.BlockSpec((1, D1), lambda i: (0, 0)),
            pl.BlockSpec((_PBLK, S1), lambda i: (i, 0)),
            pl.BlockSpec((_PBLK, S1), lambda i: (i, 0)),
            pl.BlockSpec((_PBLK, S1), lambda i: (i, 0)),
            pl.BlockSpec((_PBLK, S1), lambda i: (i, 0)),
        ],
        out_specs=[
            pl.BlockSpec((_PBLK, D1), lambda i: (i, 0)),
            pl.BlockSpec((_PBLK, 128), lambda i: (i, 0)),
        ],
        out_shape=[
            jax.ShapeDtypeStruct((N, D1), jnp.float32),
            jax.ShapeDtypeStruct((N, 128), jnp.int32),
        ],
    )(features, w1f, b1.reshape(1, D1), in_sample, out_sample, ina, outa)


def _dec_body(wd2_ref, wd1_ref, bd2_ref, bd1t_ref, w2_ref, r0_ref,
              wcatt_ref, ccatt_ref, u_ref):
    wd1 = wd1_ref[...]
    bd1t = bd1t_ref[...]
    for g in range(G):
        wcatt_ref[g * IN_DIM:(g + 1) * IN_DIM, :] = lax.dot_general(
            wd1, wd2_ref[:, g * D1:(g + 1) * D1],
            (((0,), (1,)), ((), ())),
            preferred_element_type=jnp.float32)
        cct = lax.dot_general(
            wd1, bd2_ref[:, g * D1:(g + 1) * D1],
            (((0,), (1,)), ((), ())),
            preferred_element_type=jnp.float32) + bd1t
        ccatt_ref[g * IN_DIM:(g + 1) * IN_DIM, :] = jnp.broadcast_to(
            cct, (IN_DIM, D2))
        u_ref[:, g * D1:(g + 1) * D1] = jnp.dot(
            r0_ref[...], w2_ref[g * D1:(g + 1) * D1, :],
            preferred_element_type=jnp.float32)


def _precompute_dec(w_dec2, w_dec1, b_dec2, b_dec1, w_enc2, r0):
    return pl.pallas_call(
        _dec_body,
        in_specs=[
            pl.BlockSpec((D2, G * D1), lambda: (0, 0)),
            pl.BlockSpec((D1, IN_DIM), lambda: (0, 0)),
            pl.BlockSpec((1, G * D1), lambda: (0, 0)),
            pl.BlockSpec((IN_DIM, 1), lambda: (0, 0)),
            pl.BlockSpec((G * D1, D2), lambda: (0, 0)),
            pl.BlockSpec((1, D1), lambda: (0, 0)),
        ],
        out_specs=[
            pl.BlockSpec((G * IN_DIM, D2), lambda: (0, 0)),
            pl.BlockSpec((G * IN_DIM, D2), lambda: (0, 0)),
            pl.BlockSpec((1, G * D1), lambda: (0, 0)),
        ],
        out_shape=[
            jax.ShapeDtypeStruct((G * IN_DIM, D2), jnp.float32),
            jax.ShapeDtypeStruct((G * IN_DIM, D2), jnp.float32),
            jax.ShapeDtypeStruct((1, G * D1), jnp.float32),
        ],
    )(w_dec2, w_dec1, b_dec2.reshape(1, G * D1), b_dec1.reshape(IN_DIM, 1),
      w_enc2, r0.reshape(1, D1))


def _main_body(h_ref, wf_ref, w2_ref, u_ref, b2_ref, h2_ref):
    rows = h2_ref.shape[0]
    acc = (jnp.dot(wf_ref[...], u_ref[...], preferred_element_type=jnp.float32)
           + b2_ref[...])
    h4 = h_ref[...].reshape(rows // 8, G, 8, D1)
    for g in range(G):
        hg = h4[:, g, :, :].reshape(rows, D1)
        acc = acc + jnp.dot(hg.astype(jnp.bfloat16),
                            w2_ref[g * D1:(g + 1) * D1, :].astype(
                                jnp.bfloat16),
                            preferred_element_type=jnp.float32)
    h2_ref[...] = acc


def _main(h, wfr, w_enc2, u, b_enc2):
    nblk = 16
    rows = B * 2 * S1 // nblk
    return pl.pallas_call(
        _main_body,
        grid=(nblk,),
        in_specs=[
            pl.BlockSpec((rows * G, D1), lambda i: (i, 0)),
            pl.BlockSpec((rows, G), lambda i: (i, 0)),
            pl.BlockSpec((G * D1, D2), lambda i: (0, 0)),
            pl.BlockSpec((G, D2), lambda i: (0, 0)),
            pl.BlockSpec((1, D2), lambda i: (0, 0)),
        ],
        out_specs=pl.BlockSpec((rows, D2), lambda i: (i, 0)),
        out_shape=jax.ShapeDtypeStruct((B * 2 * S1, D2), jnp.float32),
    )(h, wfr, w_enc2, u, b_enc2.reshape(1, D2))


_SPB = 4
_RBLK = _SPB * G


def _recon_body(h2_ref, wcatt_ref, ccatt_ref, rec_ref):
    i = pl.program_id(0)
    for sl in range(_SPB):
        h2s = h2_ref[:, i * _SPB + sl, :].astype(jnp.bfloat16)
        for g in range(G):
            rec_ref[:, sl * G + g, :] = (
                lax.dot_general(
                    wcatt_ref[pl.ds(g * IN_DIM, IN_DIM), :].astype(
                        jnp.bfloat16),
                    h2s, (((1,), (1,)), ((), ())),
                    preferred_element_type=jnp.float32)
                + ccatt_ref[pl.ds(g * IN_DIM, IN_DIM), :])


def _recon(h2f, wcatt, ccatt):
    nblk = (2 * S1) // _SPB
    return pl.pallas_call(
        _recon_body,
        grid=(nblk,),
        compiler_params=pltpu.CompilerParams(
            vmem_limit_bytes=100 * 1024 * 1024),
        in_specs=[
            pl.BlockSpec((B, 2 * S1, D2), lambda i: (0, 0, 0)),
            pl.BlockSpec((G * IN_DIM, D2), lambda i: (0, 0)),
            pl.BlockSpec((G * IN_DIM, D2), lambda i: (0, 0)),
        ],
        out_specs=pl.BlockSpec((IN_DIM, _RBLK, B), lambda i: (0, i, 0)),
        out_shape=jax.ShapeDtypeStruct((IN_DIM, M, B), jnp.float32),
    )(h2f.reshape(B, 2 * S1, D2), wcatt, ccatt)


def kernel(node_ids, features, in_sample, out_sample, in_sample_amnt,
           out_sample_amnt, W_enc1, b_enc1, W_enc2, b_enc2, W_dec2, b_dec2,
           W_dec1, b_dec1):
    ina = in_sample_amnt[..., 0]
    outa = out_sample_amnt[..., 0]
    r0 = W_enc1[0]

    p_tab, tab = _precompute_p(features, W_enc1[1:], b_enc1,
                               in_sample.astype(jnp.int32),
                               out_sample.astype(jnp.int32), ina, outa)
    wcatt, ccatt, u2 = _precompute_dec(W_dec2, W_dec1, b_dec2, b_dec1,
                                       W_enc2, r0)
    h, wf, wc = _sc_gather(node_ids.astype(jnp.int32), tab, p_tab)

    wfr = wf.reshape(B * 2 * S1, G)
    h2f = _main(h, wfr, W_enc2, u2.reshape(G, D1), b_enc2)
    rect = _recon(h2f, wcatt, ccatt)

    return (h2f.reshape(B, 2 * S1, D2),
            jnp.transpose(rect, (2, 1, 0)),
            wc.reshape(B, M))
